# baseline (speedup 1.0000x reference)
import functools

import jax
import jax.numpy as jnp
from jax import lax
from jax.experimental import pallas as pl
from jax.experimental.pallas import tpu as pltpu
from jax.experimental.pallas import tpu_sc as plsc

_G = 64
_NC = 2
_NS = 16
_DEPTH = 8


def _pick_chunk(epw):
    for cand in range(min(epw, 80), 0, -1):
        if epw % cand == 0 and (cand % 8 == 0 or cand < 8):
            return cand
    return 1


@functools.lru_cache(maxsize=None)
def _build_agg(n, d, nchunk, ch):
    dh = d // 2
    assert dh % 16 == 0
    align = _NS * 32
    n_pad = (n + align - 1) // align * align
    rows_per_tile = n_pad // _NS
    zr = 32
    nz = rows_per_tile // zr
    assert nchunk >= _DEPTH
    assert ch >= zr, "rows[0] doubles as the zero source"
    mesh = plsc.VectorSubcoreMesh(
        core_axis_name="c", subcore_axis_name="s",
        num_cores=_NC, num_subcores=_NS)

    nloop = nchunk // _DEPTH
    rem = nchunk - _DEPTH * nloop

    epw = nchunk * ch
    e_total = _NS * epw

    def body(h2_hbm, ei_hbm, out_hbm, src_v, *bufs):
        rows = list(bufs[:_DEPTH])
        dstb = list(bufs[_DEPTH:2 * _DEPTH])
        agg_sh = bufs[2 * _DEPTH]
        sg = list(bufs[2 * _DEPTH + 1:3 * _DEPTH + 1])
        sd = list(bufs[3 * _DEPTH + 1:4 * _DEPTH + 1])
        c = lax.axis_index("c")
        s = lax.axis_index("s")
        table = h2_hbm.at[c]
        zero = jnp.zeros((16,), jnp.float32)
        for i in range(zr):
            for j in range(dh // 16):
                rows[0][i, pl.ds(j * 16, 16)] = zero
        base = s * rows_per_tile
        for k in range(nz):
            pltpu.async_copy(rows[0].at[pl.ds(0, zr)],
                             agg_sh.at[pl.ds(base + k * zr, zr)], sg[0])
        sbase = pl.multiple_of(s * epw, 8)
        dbase = pl.multiple_of(e_total + s * epw, 8)
        pltpu.sync_copy(ei_hbm.at[pl.ds(sbase, epw)], src_v)

        def dst_slice(j):
            return ei_hbm.at[pl.ds(pl.multiple_of(dbase + j * ch, 8), ch)]

        def src_idx(j):
            return src_v.at[pl.ds(pl.multiple_of(j * ch, 8), ch)]

        for b in range(_DEPTH):
            pltpu.async_copy(dst_slice(b), dstb[b], sd[b])
        for k in range(nz):
            pltpu.make_async_copy(
                rows[0].at[pl.ds(0, zr)],
                agg_sh.at[pl.ds(base + k * zr, zr)], sg[0]).wait()
        for b in range(_DEPTH):
            pltpu.async_copy(table.at[src_idx(b)], rows[b], sg[b])
        plsc.subcore_barrier()

        def step(k, carry):
            for b in range(_DEPTH):
                j = _DEPTH * k + b
                pltpu.make_async_copy(
                    table.at[src_idx(j)], rows[b], sg[b]).wait()
                pltpu.make_async_copy(dst_slice(j), dstb[b], sd[b]).wait()
                pltpu.sync_copy(rows[b], agg_sh.at[dstb[b]], add=True)
                jc = jnp.minimum(j + _DEPTH, nchunk - 1)
                pltpu.async_copy(dst_slice(jc), dstb[b], sd[b])
                pltpu.async_copy(table.at[src_idx(jc)], rows[b], sg[b])
            return carry

        lax.fori_loop(0, nloop, step, 0)
        for b in range(_DEPTH):
            j = min(_DEPTH * nloop + b, nchunk - 1)
            pltpu.make_async_copy(table.at[src_idx(j)], rows[b], sg[b]).wait()
            pltpu.make_async_copy(dst_slice(j), dstb[b], sd[b]).wait()
            if b < rem:
                pltpu.sync_copy(rows[b], agg_sh.at[dstb[b]], add=True)
        plsc.subcore_barrier()
        pltpu.sync_copy(agg_sh.at[pl.ds(base, rows_per_tile)],
                        out_hbm.at[c, pl.ds(base, rows_per_tile)])

    return pl.kernel(
        body,
        out_type=jax.ShapeDtypeStruct((_NC, n_pad, dh), jnp.float32),
        mesh=mesh,
        compiler_params=pltpu.CompilerParams(use_tc_tiling_on_sc=False),
        scratch_types=(
            [pltpu.VMEM((nchunk * ch,), jnp.int32)]
            + [pltpu.VMEM((ch, dh), jnp.float32)] * _DEPTH
            + [pltpu.VMEM((ch,), jnp.int32)] * _DEPTH
            + [pltpu.VMEM_SHARED((n_pad, dh), jnp.float32)]
            + [pltpu.SemaphoreType.DMA] * (2 * _DEPTH)
        ),
    )


def _mlp_body(h2p_ref, aggp_ref, w1_ref, b1_ref, g_ref, be_ref, w2_ref,
              b2_ref, out_ref):
    n2 = h2p_ref.shape[1]
    n = 2.0 * n2
    d = h2p_ref.shape[2]
    dh = d // 2
    zp0 = h2p_ref[0] + aggp_ref[0, :n2]
    zp1 = h2p_ref[1] + aggp_ref[1, :n2]
    w1 = w1_ref[...]
    w1a = w1[:dh]
    w1b = w1[dh:]
    b1 = b1_ref[...]
    te = (jnp.dot(zp0[:, :dh], w1a, preferred_element_type=jnp.float32)
          + jnp.dot(zp1[:, :dh], w1b, preferred_element_type=jnp.float32)
          + b1)
    to = (jnp.dot(zp0[:, dh:], w1a, preferred_element_type=jnp.float32)
          + jnp.dot(zp1[:, dh:], w1b, preferred_element_type=jnp.float32)
          + b1)
    m = (jnp.sum(te, axis=0, keepdims=True)
         + jnp.sum(to, axis=0, keepdims=True)) / n
    v = (jnp.sum(jnp.square(te - m), axis=0, keepdims=True)
         + jnp.sum(jnp.square(to - m), axis=0, keepdims=True)) / n
    scale = lax.rsqrt(v + 1e-5) * g_ref[...]
    be = be_ref[...]
    te = jnp.maximum((te - m) * scale + be, 0.0)
    to = jnp.maximum((to - m) * scale + be, 0.0)
    w2 = w2_ref[...]
    b2 = b2_ref[...]
    ue = jnp.maximum(
        jnp.dot(te, w2, preferred_element_type=jnp.float32) + b2, 0.0)
    uo = jnp.maximum(
        jnp.dot(to, w2, preferred_element_type=jnp.float32) + b2, 0.0)
    out_ref[0] = jnp.concatenate([ue[:, :dh], uo[:, :dh]], axis=1)
    out_ref[1] = jnp.concatenate([ue[:, dh:], uo[:, dh:]], axis=1)


def _mlp_call(h2p, aggp, w1, b1, g, be, w2, b2):
    _, n2, d = h2p.shape
    return pl.pallas_call(
        _mlp_body,
        out_shape=jax.ShapeDtypeStruct((2, n2, d), jnp.float32),
    )(h2p, aggp, w1, b1.reshape(1, -1), g.reshape(1, -1), be.reshape(1, -1),
      w2, b2.reshape(1, -1))


def _head_body(h2p_ref, sege_ref, sego_ref, l1w_ref, l1b_ref, l2w_ref,
               l2b_ref, out_ref):
    n2 = h2p_ref.shape[1]
    d = h2p_ref.shape[2]
    dh = d // 2
    g_count = out_ref.shape[0]
    he = jnp.concatenate([h2p_ref[0][:, :dh], h2p_ref[1][:, :dh]], axis=1)
    ho = jnp.concatenate([h2p_ref[0][:, dh:], h2p_ref[1][:, dh:]], axis=1)
    ids = lax.broadcasted_iota(jnp.int32, (g_count, n2), 0)
    oh_e = (ids == sege_ref[...]).astype(jnp.float32)
    oh_o = (ids == sego_ref[...]).astype(jnp.float32)
    p = (jnp.dot(oh_e, he, preferred_element_type=jnp.float32)
         + jnp.dot(oh_o, ho, preferred_element_type=jnp.float32))
    p = jnp.dot(p, l1w_ref[...], preferred_element_type=jnp.float32)
    p = jnp.maximum(p + l1b_ref[...], 0.0)
    p = jnp.dot(p, l2w_ref[...], preferred_element_type=jnp.float32)
    p = p + l2b_ref[...]
    mx = jnp.max(p, axis=1, keepdims=True)
    lse = mx + jnp.log(jnp.sum(jnp.exp(p - mx), axis=1, keepdims=True))
    out_ref[...] = p - lse


def _head_call(h2p, seg, l1w, l1b, l2w, l2b):
    _, n2, d = h2p.shape
    dout = l2w.shape[1]
    l2w_p = jnp.pad(l2w, ((0, 0), (0, d - dout)))
    l2b_p = jnp.pad(l2b, (0, d - dout), constant_values=-1e30)
    seg2 = seg.reshape(n2, 2)
    out = pl.pallas_call(
        _head_body,
        out_shape=jax.ShapeDtypeStruct((_G, d), jnp.float32),
    )(h2p, seg2[:, 0].reshape(1, n2), seg2[:, 1].reshape(1, n2), l1w,
      l1b.reshape(1, -1), l2w_p, l2b_p.reshape(1, -1))
    return out[:, :dout]


def _pack_body(x_ref, out_ref):
    dh = x_ref.shape[1] // 2
    xe = x_ref[0::2]
    xo = x_ref[1::2]
    out_ref[0] = jnp.concatenate([xe[:, :dh], xo[:, :dh]], axis=1)
    out_ref[1] = jnp.concatenate([xe[:, dh:], xo[:, dh:]], axis=1)


def kernel(x, edge_index, batch, params):
    n, d = x.shape
    dh = d // 2
    n2 = n // 2
    e = edge_index.shape[1]
    epw = e // _NS
    ch = _pick_chunk(epw)
    nchunk = epw // ch

    ei_flat = edge_index.reshape(-1)

    agg_fn = _build_agg(n, d, nchunk, ch)
    n_pad = _NS * 32 * ((n + _NS * 32 - 1) // (_NS * 32))

    h2p = pl.pallas_call(
        _pack_body,
        out_shape=jax.ShapeDtypeStruct((2, n2, d), jnp.float32),
    )(x)
    for i in range(5):
        agg = agg_fn(h2p.reshape(2, n, dh), ei_flat)
        aggp = agg.reshape(2, n_pad // 2, d)
        h2p = _mlp_call(h2p, aggp, params['c%d_W1' % i], params['c%d_b1' % i],
                        params['c%d_g' % i], params['c%d_be' % i],
                        params['c%d_W2' % i], params['c%d_b2' % i])
    return _head_call(h2p, batch, params['lin1_W'], params['lin1_b'],
                      params['lin2_W'], params['lin2_b'])


# speedup vs baseline: 1.0093x; 1.0093x over previous
import functools

import jax
import jax.numpy as jnp
from jax import lax
from jax.experimental import pallas as pl
from jax.experimental.pallas import tpu as pltpu
from jax.experimental.pallas import tpu_sc as plsc

_G = 64
_NC = 2
_NS = 16
_DEPTH = 6


def _pick_chunk(epw):
    for cand in range(min(epw, 80), 0, -1):
        if epw % cand == 0 and (cand % 8 == 0 or cand < 8):
            return cand
    return 1


@functools.lru_cache(maxsize=None)
def _build_agg(n, d, nchunk, ch):
    dh = d // 2
    assert dh % 16 == 0
    align = _NS * 32
    n_pad = (n + align - 1) // align * align
    rows_per_tile = n_pad // _NS
    zr = 32
    nz = rows_per_tile // zr
    assert nchunk >= _DEPTH
    assert ch >= zr, "rows[0] doubles as the zero source"
    mesh = plsc.VectorSubcoreMesh(
        core_axis_name="c", subcore_axis_name="s",
        num_cores=_NC, num_subcores=_NS)

    nloop = nchunk // _DEPTH
    rem = nchunk - _DEPTH * nloop

    epw = nchunk * ch
    e_total = _NS * epw

    def body(h2_hbm, ei_hbm, out_hbm, src_v, *bufs):
        rows = list(bufs[:_DEPTH])
        dstb = list(bufs[_DEPTH:2 * _DEPTH])
        agg_sh = bufs[2 * _DEPTH]
        sg = list(bufs[2 * _DEPTH + 1:3 * _DEPTH + 1])
        sd = list(bufs[3 * _DEPTH + 1:4 * _DEPTH + 1])
        c = lax.axis_index("c")
        s = lax.axis_index("s")
        table = h2_hbm.at[c]
        zero = jnp.zeros((16,), jnp.float32)
        for i in range(zr):
            for j in range(dh // 16):
                rows[0][i, pl.ds(j * 16, 16)] = zero
        base = s * rows_per_tile
        for k in range(nz):
            pltpu.async_copy(rows[0].at[pl.ds(0, zr)],
                             agg_sh.at[pl.ds(base + k * zr, zr)], sg[0])
        sbase = pl.multiple_of(s * epw, 8)
        dbase = pl.multiple_of(e_total + s * epw, 8)
        pltpu.sync_copy(ei_hbm.at[pl.ds(sbase, epw)], src_v)

        def dst_slice(j):
            return ei_hbm.at[pl.ds(pl.multiple_of(dbase + j * ch, 8), ch)]

        def src_idx(j):
            return src_v.at[pl.ds(pl.multiple_of(j * ch, 8), ch)]

        for b in range(_DEPTH):
            pltpu.async_copy(dst_slice(b), dstb[b], sd[b])
        for k in range(nz):
            pltpu.make_async_copy(
                rows[0].at[pl.ds(0, zr)],
                agg_sh.at[pl.ds(base + k * zr, zr)], sg[0]).wait()
        for b in range(_DEPTH):
            pltpu.async_copy(table.at[src_idx(b)], rows[b], sg[b])
        plsc.subcore_barrier()

        def step(k, carry):
            for b in range(_DEPTH):
                j = _DEPTH * k + b
                pltpu.make_async_copy(
                    table.at[src_idx(j)], rows[b], sg[b]).wait()
                pltpu.make_async_copy(dst_slice(j), dstb[b], sd[b]).wait()
                pltpu.sync_copy(rows[b], agg_sh.at[dstb[b]], add=True)
                jc = jnp.minimum(j + _DEPTH, nchunk - 1)
                pltpu.async_copy(dst_slice(jc), dstb[b], sd[b])
                pltpu.async_copy(table.at[src_idx(jc)], rows[b], sg[b])
            return carry

        lax.fori_loop(0, nloop, step, 0)
        for b in range(_DEPTH):
            j = min(_DEPTH * nloop + b, nchunk - 1)
            pltpu.make_async_copy(table.at[src_idx(j)], rows[b], sg[b]).wait()
            pltpu.make_async_copy(dst_slice(j), dstb[b], sd[b]).wait()
            if b < rem:
                pltpu.sync_copy(rows[b], agg_sh.at[dstb[b]], add=True)
        plsc.subcore_barrier()
        pltpu.sync_copy(agg_sh.at[pl.ds(base, rows_per_tile)],
                        out_hbm.at[c, pl.ds(base, rows_per_tile)])

    return pl.kernel(
        body,
        out_type=jax.ShapeDtypeStruct((_NC, n_pad, dh), jnp.float32),
        mesh=mesh,
        compiler_params=pltpu.CompilerParams(use_tc_tiling_on_sc=False),
        scratch_types=(
            [pltpu.VMEM((nchunk * ch,), jnp.int32)]
            + [pltpu.VMEM((ch, dh), jnp.float32)] * _DEPTH
            + [pltpu.VMEM((ch,), jnp.int32)] * _DEPTH
            + [pltpu.VMEM_SHARED((n_pad, dh), jnp.float32)]
            + [pltpu.SemaphoreType.DMA] * (2 * _DEPTH)
        ),
    )


def _mlp_body(h2p_ref, aggp_ref, w1_ref, b1_ref, g_ref, be_ref, w2_ref,
              b2_ref, out_ref):
    n2 = h2p_ref.shape[1]
    n = 2.0 * n2
    d = h2p_ref.shape[2]
    dh = d // 2
    zp0 = h2p_ref[0] + aggp_ref[0, :n2]
    zp1 = h2p_ref[1] + aggp_ref[1, :n2]
    w1 = w1_ref[...]
    w1a = w1[:dh]
    w1b = w1[dh:]
    b1 = b1_ref[...]
    te = (jnp.dot(zp0[:, :dh], w1a, preferred_element_type=jnp.float32)
          + jnp.dot(zp1[:, :dh], w1b, preferred_element_type=jnp.float32)
          + b1)
    to = (jnp.dot(zp0[:, dh:], w1a, preferred_element_type=jnp.float32)
          + jnp.dot(zp1[:, dh:], w1b, preferred_element_type=jnp.float32)
          + b1)
    m = (jnp.sum(te, axis=0, keepdims=True)
         + jnp.sum(to, axis=0, keepdims=True)) / n
    v = (jnp.sum(jnp.square(te - m), axis=0, keepdims=True)
         + jnp.sum(jnp.square(to - m), axis=0, keepdims=True)) / n
    scale = lax.rsqrt(v + 1e-5) * g_ref[...]
    be = be_ref[...]
    te = jnp.maximum((te - m) * scale + be, 0.0)
    to = jnp.maximum((to - m) * scale + be, 0.0)
    w2 = w2_ref[...]
    b2 = b2_ref[...]
    ue = jnp.maximum(
        jnp.dot(te, w2, preferred_element_type=jnp.float32) + b2, 0.0)
    uo = jnp.maximum(
        jnp.dot(to, w2, preferred_element_type=jnp.float32) + b2, 0.0)
    out_ref[0] = jnp.concatenate([ue[:, :dh], uo[:, :dh]], axis=1)
    out_ref[1] = jnp.concatenate([ue[:, dh:], uo[:, dh:]], axis=1)


def _mlp_call(h2p, aggp, w1, b1, g, be, w2, b2):
    _, n2, d = h2p.shape
    return pl.pallas_call(
        _mlp_body,
        out_shape=jax.ShapeDtypeStruct((2, n2, d), jnp.float32),
    )(h2p, aggp, w1, b1.reshape(1, -1), g.reshape(1, -1), be.reshape(1, -1),
      w2, b2.reshape(1, -1))


def _head_body(h2p_ref, sege_ref, sego_ref, l1w_ref, l1b_ref, l2w_ref,
               l2b_ref, out_ref):
    n2 = h2p_ref.shape[1]
    d = h2p_ref.shape[2]
    dh = d // 2
    g_count = out_ref.shape[0]
    he = jnp.concatenate([h2p_ref[0][:, :dh], h2p_ref[1][:, :dh]], axis=1)
    ho = jnp.concatenate([h2p_ref[0][:, dh:], h2p_ref[1][:, dh:]], axis=1)
    ids = lax.broadcasted_iota(jnp.int32, (g_count, n2), 0)
    oh_e = (ids == sege_ref[...]).astype(jnp.float32)
    oh_o = (ids == sego_ref[...]).astype(jnp.float32)
    p = (jnp.dot(oh_e, he, preferred_element_type=jnp.float32)
         + jnp.dot(oh_o, ho, preferred_element_type=jnp.float32))
    p = jnp.dot(p, l1w_ref[...], preferred_element_type=jnp.float32)
    p = jnp.maximum(p + l1b_ref[...], 0.0)
    p = jnp.dot(p, l2w_ref[...], preferred_element_type=jnp.float32)
    p = p + l2b_ref[...]
    mx = jnp.max(p, axis=1, keepdims=True)
    lse = mx + jnp.log(jnp.sum(jnp.exp(p - mx), axis=1, keepdims=True))
    out_ref[...] = p - lse


def _head_call(h2p, seg, l1w, l1b, l2w, l2b):
    _, n2, d = h2p.shape
    dout = l2w.shape[1]
    l2w_p = jnp.pad(l2w, ((0, 0), (0, d - dout)))
    l2b_p = jnp.pad(l2b, (0, d - dout), constant_values=-1e30)
    seg2 = seg.reshape(n2, 2)
    out = pl.pallas_call(
        _head_body,
        out_shape=jax.ShapeDtypeStruct((_G, d), jnp.float32),
    )(h2p, seg2[:, 0].reshape(1, n2), seg2[:, 1].reshape(1, n2), l1w,
      l1b.reshape(1, -1), l2w_p, l2b_p.reshape(1, -1))
    return out[:, :dout]


def _pack_body(x_ref, out_ref):
    dh = x_ref.shape[1] // 2
    xe = x_ref[0::2]
    xo = x_ref[1::2]
    out_ref[0] = jnp.concatenate([xe[:, :dh], xo[:, :dh]], axis=1)
    out_ref[1] = jnp.concatenate([xe[:, dh:], xo[:, dh:]], axis=1)


def kernel(x, edge_index, batch, params):
    n, d = x.shape
    dh = d // 2
    n2 = n // 2
    e = edge_index.shape[1]
    epw = e // _NS
    ch = _pick_chunk(epw)
    nchunk = epw // ch

    ei_flat = edge_index.reshape(-1)

    agg_fn = _build_agg(n, d, nchunk, ch)
    n_pad = _NS * 32 * ((n + _NS * 32 - 1) // (_NS * 32))

    h2p = pl.pallas_call(
        _pack_body,
        out_shape=jax.ShapeDtypeStruct((2, n2, d), jnp.float32),
    )(x)
    for i in range(5):
        agg = agg_fn(h2p.reshape(2, n, dh), ei_flat)
        aggp = agg.reshape(2, n_pad // 2, d)
        h2p = _mlp_call(h2p, aggp, params['c%d_W1' % i], params['c%d_b1' % i],
                        params['c%d_g' % i], params['c%d_be' % i],
                        params['c%d_W2' % i], params['c%d_b2' % i])
    return _head_call(h2p, batch, params['lin1_W'], params['lin1_b'],
                      params['lin2_W'], params['lin2_b'])
